# baseline (speedup 1.0000x reference)
import functools

import jax
import jax.numpy as jnp
from jax import lax
from jax.experimental import pallas as pl
from jax.experimental.pallas import tpu as pltpu
from jax.experimental.pallas import tpu_sc as plsc

BATCH = 16384
HIST = 50
HIDDEN = 64
B_TOTAL = BATCH * HIST

NC = 2
NS = 16
NW = NC * NS
CHUNK = 128
N_CHUNKS_TOTAL = B_TOTAL // CHUNK
C_PER_W = N_CHUNKS_TOTAL // NW
N_GROUPS = C_PER_W // 2

_mesh = plsc.VectorSubcoreMesh(core_axis_name="c", subcore_axis_name="s")


@functools.partial(
    pl.kernel,
    mesh=_mesh,
    out_type=jax.ShapeDtypeStruct((HIST, HIDDEN, BATCH), jnp.float32),
    scratch_types=[
        pltpu.VMEM((C_PER_W, CHUNK), jnp.int32),
        pltpu.VMEM((C_PER_W, CHUNK), jnp.int32),
        pltpu.VMEM((2, CHUNK, 128), jnp.float32),
        pltpu.VMEM((2, HIDDEN, CHUNK), jnp.float32),
        pltpu.SemaphoreType.DMA,
        pltpu.SemaphoreType.DMA,
        pltpu.SemaphoreType.DMA,
        pltpu.SemaphoreType.DMA,
    ],
    compiler_params=pltpu.CompilerParams(
        use_tc_tiling_on_sc=True, needs_layout_passes=False
    ),
)
def _sc_gather(table2_hbm, idx_hbm, out_hbm, pid_v, half_v, gbuf, obuf,
               gsem0, gsem1, osem0, osem1):
    gsems = (gsem0, gsem1)
    osems = (osem0, osem1)
    wid = lax.axis_index("s") * NC + lax.axis_index("c")
    base_cid = wid * C_PER_W

    pltpu.sync_copy(idx_hbm.at[pl.ds(base_cid, C_PER_W)], pid_v)

    def prep_body(j, _):
        for t in range(8):
            v = pid_v[j, pl.ds(16 * t, 16)]
            pid_v[j, pl.ds(16 * t, 16)] = lax.shift_right_logical(v, 1)
            half_v[j, pl.ds(16 * t, 16)] = (v & 1) * HIDDEN
        return 0

    lax.fori_loop(0, C_PER_W, prep_body, 0)

    def start_gather(j, slot):
        pltpu.async_copy(table2_hbm.at[pid_v.at[j]], gbuf.at[slot], gsems[slot])

    def wait_gather(j, slot):
        pltpu.make_async_copy(
            table2_hbm.at[pid_v.at[j]], gbuf.at[slot], gsems[slot]
        ).wait()

    start_gather(0, 0)

    rows = [lax.iota(jnp.int32, 16) + 16 * t for t in range(8)]

    def group_body(g, _):
        for b in range(2):
            j = 2 * g + b
            nj = j + 1

            @pl.when(nj < C_PER_W)
            def _():
                start_gather(nj, 1 - b)

            wait_gather(j, b)

            cid = base_cid + j
            h = lax.shift_right_logical(cid, 7)
            tb = cid & 127

            @pl.when(j >= 2)
            def _():
                pltpu.make_async_copy(
                    obuf.at[b],
                    out_hbm.at[h, :, pl.ds(tb * CHUNK, CHUNK)],
                    osems[b],
                ).wait()

            halves = [half_v[j, pl.ds(16 * t, 16)] for t in range(8)]

            def col_body(c, carry):
                for t in range(8):
                    col = carry[t] + c
                    val = plsc.load_gather(gbuf.at[b], [rows[t], col])
                    obuf[b, c, pl.ds(16 * t, 16)] = val
                return carry

            lax.fori_loop(0, HIDDEN, col_body, tuple(halves))

            pltpu.async_copy(
                obuf.at[b],
                out_hbm.at[h, :, pl.ds(tb * CHUNK, CHUNK)],
                osems[b],
            )
        return 0

    lax.fori_loop(0, N_GROUPS, group_body, 0)

    for b in range(2):
        j = C_PER_W - 2 + b
        cid = base_cid + j
        h = lax.shift_right_logical(cid, 7)
        tb = cid & 127
        pltpu.make_async_copy(
            obuf.at[b],
            out_hbm.at[h, :, pl.ds(tb * CHUNK, CHUNK)],
            osems[b],
        ).wait()


def kernel(table, item_ids):
    table2 = table.reshape(500000, 128)
    idx2 = item_ids.T.reshape(N_CHUNKS_TOTAL, CHUNK).astype(jnp.int32)
    out = _sc_gather(table2, idx2)
    return jnp.transpose(out, (2, 0, 1))


# speedup vs baseline: 1.4583x; 1.4583x over previous
import functools

import jax
import jax.numpy as jnp
from jax import lax
from jax.experimental import pallas as pl
from jax.experimental.pallas import tpu as pltpu
from jax.experimental.pallas import tpu_sc as plsc

BATCH = 16384
HIST = 50
HIDDEN = 64
B_TOTAL = BATCH * HIST

NC = 2
NS = 16
NW = NC * NS
CHUNK = 128
N_CHUNKS_TOTAL = B_TOTAL // CHUNK
C_PER_W = N_CHUNKS_TOTAL // NW
N_GROUPS = C_PER_W // 2

_mesh = plsc.VectorSubcoreMesh(core_axis_name="c", subcore_axis_name="s")


@functools.partial(
    pl.kernel,
    mesh=_mesh,
    out_type=jax.ShapeDtypeStruct((HIST, HIDDEN, BATCH), jnp.float32),
    scratch_types=[
        pltpu.VMEM((C_PER_W, CHUNK), jnp.int32),
        pltpu.VMEM((C_PER_W, CHUNK), jnp.int32),
        pltpu.VMEM((2, CHUNK, 128), jnp.float32),
        pltpu.VMEM((2, HIDDEN, CHUNK), jnp.float32),
        pltpu.SemaphoreType.DMA,
        pltpu.SemaphoreType.DMA,
        pltpu.SemaphoreType.DMA,
        pltpu.SemaphoreType.DMA,
    ],
    compiler_params=pltpu.CompilerParams(
        use_tc_tiling_on_sc=True, needs_layout_passes=False
    ),
)
def _sc_gather(table2_hbm, idx_hbm, out_hbm, pid_v, half_v, gbuf, obuf,
               gsem0, gsem1, osem0, osem1):
    gsems = (gsem0, gsem1)
    osems = (osem0, osem1)
    wid = lax.axis_index("s") * NC + lax.axis_index("c")
    base_cid = wid * C_PER_W

    pltpu.sync_copy(idx_hbm.at[pl.ds(base_cid, C_PER_W)], pid_v)

    def prep_body(j, _):
        for t in range(8):
            v = pid_v[j, pl.ds(16 * t, 16)]
            pid_v[j, pl.ds(16 * t, 16)] = lax.shift_right_logical(v, 1)
            half_v[j, pl.ds(16 * t, 16)] = (v & 1) * HIDDEN
        return 0

    lax.fori_loop(0, C_PER_W, prep_body, 0)

    def start_gather(j, slot):
        pltpu.async_copy(table2_hbm.at[pid_v.at[j]], gbuf.at[slot], gsems[slot])

    def wait_gather(j, slot):
        pltpu.make_async_copy(
            table2_hbm.at[pid_v.at[j]], gbuf.at[slot], gsems[slot]
        ).wait()

    start_gather(0, 0)

    rows = [lax.iota(jnp.int32, 16) + 16 * t for t in range(8)]

    def group_body(g, _):
        for b in range(2):
            j = 2 * g + b
            nj = j + 1

            @pl.when(nj < C_PER_W)
            def _():
                start_gather(nj, 1 - b)

            wait_gather(j, b)

            cid = base_cid + j
            h = lax.shift_right_logical(cid, 7)
            tb = cid & 127

            @pl.when(j >= 2)
            def _():
                pltpu.make_async_copy(
                    obuf.at[b],
                    out_hbm.at[h, :, pl.ds(tb * CHUNK, CHUNK)],
                    osems[b],
                ).wait()

            halves = [half_v[j, pl.ds(16 * t, 16)] for t in range(8)]

            @plsc.parallel_loop(0, HIDDEN, step=1, unroll=8)
            def col_body(c):
                for t in range(8):
                    col = halves[t] + c
                    val = plsc.load_gather(gbuf.at[b], [rows[t], col])
                    obuf[b, c, pl.ds(16 * t, 16)] = val

            pltpu.async_copy(
                obuf.at[b],
                out_hbm.at[h, :, pl.ds(tb * CHUNK, CHUNK)],
                osems[b],
            )
        return 0

    lax.fori_loop(0, N_GROUPS, group_body, 0)

    for b in range(2):
        j = C_PER_W - 2 + b
        cid = base_cid + j
        h = lax.shift_right_logical(cid, 7)
        tb = cid & 127
        pltpu.make_async_copy(
            obuf.at[b],
            out_hbm.at[h, :, pl.ds(tb * CHUNK, CHUNK)],
            osems[b],
        ).wait()


def kernel(table, item_ids):
    table2 = table.reshape(500000, 128)
    idx2 = item_ids.T.reshape(N_CHUNKS_TOTAL, CHUNK).astype(jnp.int32)
    out = _sc_gather(table2, idx2)
    return jnp.transpose(out, (2, 0, 1))


# speedup vs baseline: 2.1428x; 1.4694x over previous
import functools

import jax
import jax.numpy as jnp
from jax import lax
from jax.experimental import pallas as pl
from jax.experimental.pallas import tpu as pltpu
from jax.experimental.pallas import tpu_sc as plsc

BATCH = 16384
HIST = 50
HIDDEN = 64
B_TOTAL = BATCH * HIST

NC = 2
NS = 16
NW = NC * NS
CHUNK = 128
N_CHUNKS_TOTAL = B_TOTAL // CHUNK
C_PER_W = N_CHUNKS_TOTAL // NW
N_GROUPS = C_PER_W // 2

_mesh = plsc.VectorSubcoreMesh(core_axis_name="c", subcore_axis_name="s")


@functools.partial(
    pl.kernel,
    mesh=_mesh,
    out_type=jax.ShapeDtypeStruct((HIST, HIDDEN, BATCH), jnp.float32),
    scratch_types=[
        pltpu.VMEM((C_PER_W, CHUNK), jnp.int32),
        pltpu.VMEM((C_PER_W, CHUNK), jnp.int32),
        pltpu.VMEM((2, CHUNK, 128), jnp.float32),
        pltpu.VMEM((2, HIDDEN, CHUNK), jnp.float32),
        pltpu.SemaphoreType.DMA,
        pltpu.SemaphoreType.DMA,
        pltpu.SemaphoreType.DMA,
        pltpu.SemaphoreType.DMA,
    ],
    compiler_params=pltpu.CompilerParams(
        use_tc_tiling_on_sc=True, needs_layout_passes=False
    ),
)
def _sc_gather(table2_hbm, idx_hbm, out_hbm, pid_v, half_v, gbuf, obuf,
               gsem0, gsem1, osem0, osem1):
    gsems = (gsem0, gsem1)
    osems = (osem0, osem1)
    wid = lax.axis_index("s") * NC + lax.axis_index("c")
    base_cid = wid * C_PER_W

    pltpu.sync_copy(idx_hbm.at[pl.ds(base_cid, C_PER_W)], pid_v)

    def prep_body(j, _):
        for t in range(8):
            v = pid_v[j, pl.ds(16 * t, 16)]
            pid_v[j, pl.ds(16 * t, 16)] = lax.shift_right_logical(v, 1)
            half_v[j, pl.ds(16 * t, 16)] = (v & 1) * HIDDEN
        return 0

    lax.fori_loop(0, C_PER_W, prep_body, 0)

    def start_gather(j, slot):
        pltpu.async_copy(table2_hbm.at[pid_v.at[j]], gbuf.at[slot], gsems[slot])

    def wait_gather(j, slot):
        pltpu.make_async_copy(
            table2_hbm.at[pid_v.at[j]], gbuf.at[slot], gsems[slot]
        ).wait()

    start_gather(0, 0)

    rows = [lax.iota(jnp.int32, 16) + 16 * t for t in range(8)]

    def group_body(g, _):
        for b in range(2):
            j = 2 * g + b
            nj = j + 1

            @pl.when(nj < C_PER_W)
            def _():
                start_gather(nj, 1 - b)

            wait_gather(j, b)

            cid = base_cid + j
            h = lax.shift_right_logical(cid, 7)
            tb = cid & 127

            @pl.when(j >= 2)
            def _():
                pltpu.make_async_copy(
                    obuf.at[b],
                    out_hbm.at[h, :, pl.ds(tb * CHUNK, CHUNK)],
                    osems[b],
                ).wait()

            halves = [half_v[j, pl.ds(16 * t, 16)] for t in range(8)]

            lane = rows[0]

            @plsc.parallel_loop(0, HIDDEN, step=1, unroll=8)
            def col_body(c):
                svec = (c + lane) & 63
                for t in range(8):
                    val = plsc.load_gather(
                        gbuf.at[b], [rows[t], halves[t] + svec]
                    )
                    plsc.store_scatter(obuf.at[b], [svec, rows[t]], val)

            pltpu.async_copy(
                obuf.at[b],
                out_hbm.at[h, :, pl.ds(tb * CHUNK, CHUNK)],
                osems[b],
            )
        return 0

    lax.fori_loop(0, N_GROUPS, group_body, 0)

    for b in range(2):
        j = C_PER_W - 2 + b
        cid = base_cid + j
        h = lax.shift_right_logical(cid, 7)
        tb = cid & 127
        pltpu.make_async_copy(
            obuf.at[b],
            out_hbm.at[h, :, pl.ds(tb * CHUNK, CHUNK)],
            osems[b],
        ).wait()


def kernel(table, item_ids):
    table2 = table.reshape(500000, 128)
    idx2 = item_ids.T.reshape(N_CHUNKS_TOTAL, CHUNK).astype(jnp.int32)
    out = _sc_gather(table2, idx2)
    return jnp.transpose(out, (2, 0, 1))


# speedup vs baseline: 3.7395x; 1.7451x over previous
import functools

import jax
import jax.numpy as jnp
from jax import lax
from jax.experimental import pallas as pl
from jax.experimental.pallas import tpu as pltpu
from jax.experimental.pallas import tpu_sc as plsc

BATCH = 16384
HIST = 50
HIDDEN = 64
B_TOTAL = BATCH * HIST

NC = 2
NS = 16
NW = NC * NS
CHUNK = 128
N_CHUNKS_TOTAL = B_TOTAL // CHUNK
C_PER_W = N_CHUNKS_TOTAL // NW
N_GROUPS = C_PER_W // 2

_mesh = plsc.VectorSubcoreMesh(core_axis_name="c", subcore_axis_name="s")


@functools.partial(
    pl.kernel,
    mesh=_mesh,
    out_type=jax.ShapeDtypeStruct((HIST, HIDDEN, BATCH), jnp.float32),
    scratch_types=[
        pltpu.VMEM((C_PER_W, CHUNK), jnp.int32),
        pltpu.VMEM((C_PER_W, CHUNK), jnp.int32),
        pltpu.VMEM((2, CHUNK, 128), jnp.float32),
        pltpu.VMEM((2, HIDDEN, CHUNK), jnp.float32),
        pltpu.SemaphoreType.DMA,
        pltpu.SemaphoreType.DMA,
        pltpu.SemaphoreType.DMA,
        pltpu.SemaphoreType.DMA,
    ],
    compiler_params=pltpu.CompilerParams(
        use_tc_tiling_on_sc=True, needs_layout_passes=False
    ),
)
def _sc_gather(table2_hbm, idx_hbm, out_hbm, pid_v, half_v, gbuf, obuf,
               gsem0, gsem1, osem0, osem1):
    gsems = (gsem0, gsem1)
    osems = (osem0, osem1)
    wid = lax.axis_index("s") * NC + lax.axis_index("c")
    base_cid = wid * C_PER_W

    pltpu.sync_copy(idx_hbm.at[pl.ds(base_cid, C_PER_W)], pid_v)

    def prep_body(j, _):
        for t in range(8):
            v = pid_v[j, pl.ds(16 * t, 16)]
            pid_v[j, pl.ds(16 * t, 16)] = lax.shift_right_logical(v, 1)
            half_v[j, pl.ds(16 * t, 16)] = (v & 1) * HIDDEN
        return 0

    lax.fori_loop(0, C_PER_W, prep_body, 0)

    def start_gather(j, slot):
        pltpu.async_copy(table2_hbm.at[pid_v.at[j]], gbuf.at[slot], gsems[slot])

    def wait_gather(j, slot):
        pltpu.make_async_copy(
            table2_hbm.at[pid_v.at[j]], gbuf.at[slot], gsems[slot]
        ).wait()

    start_gather(0, 0)

    rows = [lax.iota(jnp.int32, 16) + 16 * t for t in range(8)]

    def group_body(g, _):
        for b in range(2):
            j = 2 * g + b
            nj = j + 1

            @pl.when(nj < C_PER_W)
            def _():
                start_gather(nj, 1 - b)

            wait_gather(j, b)

            cid = base_cid + j
            h = lax.shift_right_logical(cid, 7)
            tb = cid & 127

            @pl.when(j >= 2)
            def _():
                pltpu.make_async_copy(
                    obuf.at[b],
                    out_hbm.at[h, :, pl.ds(tb * CHUNK, CHUNK)],
                    osems[b],
                ).wait()

            halves = [half_v[j, pl.ds(16 * t, 16)] for t in range(8)]

            lane = rows[0]

            @plsc.parallel_loop(0, HIDDEN, step=1, unroll=8)
            def col_body(c):
                svec = (c + lane) & 63
                for t in range(8):
                    val = plsc.load_gather(
                        gbuf.at[b], [rows[t], halves[t] + svec]
                    )
                    plsc.store_scatter(obuf.at[b], [svec, rows[t]], val)

            pltpu.async_copy(
                obuf.at[b],
                out_hbm.at[h, :, pl.ds(tb * CHUNK, CHUNK)],
                osems[b],
            )
        return 0

    lax.fori_loop(0, N_GROUPS, group_body, 0)

    for b in range(2):
        j = C_PER_W - 2 + b
        cid = base_cid + j
        h = lax.shift_right_logical(cid, 7)
        tb = cid & 127
        pltpu.make_async_copy(
            obuf.at[b],
            out_hbm.at[h, :, pl.ds(tb * CHUNK, CHUNK)],
            osems[b],
        ).wait()


N_BLK = 7812
N_MAIN = 244


@functools.partial(
    pl.kernel,
    mesh=_mesh,
    out_type=jax.ShapeDtypeStruct((500000, 128), jnp.float32),
    scratch_types=[
        pltpu.VMEM((2, HIDDEN, CHUNK), jnp.float32),
        pltpu.VMEM((2, HIDDEN, CHUNK), jnp.float32),
        pltpu.VMEM((32, CHUNK), jnp.float32),
        pltpu.SemaphoreType.DMA,
        pltpu.SemaphoreType.DMA,
        pltpu.SemaphoreType.DMA,
        pltpu.SemaphoreType.DMA,
    ],
    compiler_params=pltpu.CompilerParams(
        use_tc_tiling_on_sc=True, needs_layout_passes=False
    ),
)
def _sc_pack(tabt_hbm, tail_hbm, out_hbm, gbuf, obuf, tailv,
             gsem0, gsem1, osem0, osem1):
    gsems = (gsem0, gsem1)
    osems = (osem0, osem1)
    wid = lax.axis_index("s") * NC + lax.axis_index("c")

    lane = lax.iota(jnp.int32, 16)
    hi64 = (lane & 1) << 6
    rows = [lane + 16 * t for t in range(8)]
    rowhalf = [(lane >> 1) + 8 * t for t in range(8)]

    def start(k, slot):
        pltpu.async_copy(
            tabt_hbm.at[:, pl.ds(k * CHUNK, CHUNK)], gbuf.at[slot], gsems[slot]
        )

    def wait_in(k, slot):
        pltpu.make_async_copy(
            tabt_hbm.at[:, pl.ds(k * CHUNK, CHUNK)], gbuf.at[slot], gsems[slot]
        ).wait()

    def transpose_block(b):
        @plsc.parallel_loop(0, HIDDEN, step=1, unroll=8)
        def col_body(c):
            svec = (c + lane) & 63
            colv = svec + hi64
            for t in range(8):
                val = plsc.load_gather(gbuf.at[b], [svec, rows[t]])
                plsc.store_scatter(obuf.at[b], [rowhalf[t], colv], val)

    def out_slice(k):
        return out_hbm.at[pl.ds(k * HIDDEN, HIDDEN)]

    start(wid, 0)

    def group_body(grp, _):
        for b in range(2):
            g = 2 * grp + b
            k = 32 * g + wid

            @pl.when(g < N_MAIN - 1)
            def _():
                start(k + 32, 1 - b)

            wait_in(k, b)

            @pl.when(g >= 2)
            def _():
                pltpu.make_async_copy(
                    obuf.at[b], out_slice(k - 64), osems[b]
                ).wait()

            transpose_block(b)
            pltpu.async_copy(obuf.at[b], out_slice(k), osems[b])
        return 0

    lax.fori_loop(0, N_MAIN // 2, group_body, 0)

    for b in range(2):
        k_last = 32 * (N_MAIN - 2 + b) + wid
        pltpu.make_async_copy(obuf.at[b], out_slice(k_last), osems[b]).wait()

    @pl.when(wid < 4)
    def _():
        k = N_MAIN * 32 + wid
        pltpu.sync_copy(tabt_hbm.at[:, pl.ds(k * CHUNK, CHUNK)], gbuf.at[0])
        transpose_block(0)
        pltpu.sync_copy(obuf.at[0], out_slice(k))

    @pl.when(wid == 31)
    def _():
        pltpu.sync_copy(tail_hbm, tailv)
        pltpu.sync_copy(tailv, out_hbm.at[pl.ds(N_BLK * HIDDEN, 32)])


def kernel(table, item_ids):
    tablet = table.T
    tail = table[N_BLK * CHUNK:].reshape(32, 128)
    table2 = _sc_pack(tablet, tail)
    idx2 = item_ids.T.reshape(N_CHUNKS_TOTAL, CHUNK).astype(jnp.int32)
    out = _sc_gather(table2, idx2)
    return jnp.transpose(out, (2, 0, 1))


# speedup vs baseline: 4.3411x; 1.1609x over previous
import functools

import jax
import jax.numpy as jnp
from jax import lax
from jax.experimental import pallas as pl
from jax.experimental.pallas import tpu as pltpu
from jax.experimental.pallas import tpu_sc as plsc

BATCH = 16384
HIST = 50
HIDDEN = 64
B_TOTAL = BATCH * HIST

NC = 2
NS = 16
NW = NC * NS
CHUNK = 128
N_CHUNKS_TOTAL = B_TOTAL // CHUNK
C_PER_W = N_CHUNKS_TOTAL // NW
N_GROUPS = C_PER_W // 2

_mesh = plsc.VectorSubcoreMesh(core_axis_name="c", subcore_axis_name="s")


RING = 4


@functools.partial(
    pl.kernel,
    mesh=_mesh,
    out_type=jax.ShapeDtypeStruct((HIST, HIDDEN, BATCH), jnp.float32),
    scratch_types=[
        pltpu.VMEM((C_PER_W, CHUNK), jnp.int32),
        pltpu.VMEM((RING, CHUNK), jnp.int32),
        pltpu.VMEM((RING, CHUNK, 128), jnp.float32),
        pltpu.VMEM((2, HIDDEN, CHUNK), jnp.float32),
        pltpu.SemaphoreType.DMA,
        pltpu.SemaphoreType.DMA,
        pltpu.SemaphoreType.DMA,
        pltpu.SemaphoreType.DMA,
        pltpu.SemaphoreType.DMA,
        pltpu.SemaphoreType.DMA,
    ],
    compiler_params=pltpu.CompilerParams(
        use_tc_tiling_on_sc=True, needs_layout_passes=False
    ),
)
def _sc_gather(table2_hbm, idx_hbm, out_hbm, idx_v, pid_v, gbuf, obuf,
               gsem0, gsem1, gsem2, gsem3, osem0, osem1):
    gsems = (gsem0, gsem1, gsem2, gsem3)
    osems = (osem0, osem1)
    wid = lax.axis_index("s") * NC + lax.axis_index("c")
    base_cid = wid * C_PER_W

    pltpu.sync_copy(idx_hbm.at[pl.ds(base_cid, C_PER_W)], idx_v)

    rows = [lax.iota(jnp.int32, 16) + 16 * t for t in range(8)]
    lane = rows[0]

    def start_gather(j, slot):
        for t in range(8):
            v = idx_v[j, pl.ds(16 * t, 16)]
            pid_v[slot, pl.ds(16 * t, 16)] = lax.shift_right_logical(v, 1)
        pltpu.async_copy(
            table2_hbm.at[pid_v.at[slot]], gbuf.at[slot], gsems[slot]
        )

    def wait_gather(slot):
        pltpu.make_async_copy(
            table2_hbm.at[pid_v.at[slot]], gbuf.at[slot], gsems[slot]
        ).wait()

    for j0 in range(RING - 1):
        start_gather(j0, j0)

    def group_body(g, _):
        for b in range(RING):
            j = RING * g + b
            ob = b & 1
            nj = j + RING - 1

            @pl.when(nj < C_PER_W)
            def _():
                start_gather(nj, (b + RING - 1) % RING)

            wait_gather(b)

            cid = base_cid + j
            h = lax.shift_right_logical(cid, 7)
            tb = cid & 127

            @pl.when(j >= 2)
            def _():
                pltpu.make_async_copy(
                    obuf.at[ob],
                    out_hbm.at[h, :, pl.ds(tb * CHUNK, CHUNK)],
                    osems[ob],
                ).wait()

            halves = [(idx_v[j, pl.ds(16 * t, 16)] & 1) << 6 for t in range(8)]

            @plsc.parallel_loop(0, HIDDEN, step=1, unroll=8)
            def col_body(c):
                svec = (c + lane) & 63
                for t in range(8):
                    val = plsc.load_gather(
                        gbuf.at[b], [rows[t], halves[t] + svec]
                    )
                    plsc.store_scatter(obuf.at[ob], [svec, rows[t]], val)

            pltpu.async_copy(
                obuf.at[ob],
                out_hbm.at[h, :, pl.ds(tb * CHUNK, CHUNK)],
                osems[ob],
            )
        return 0

    lax.fori_loop(0, C_PER_W // RING, group_body, 0)

    for b in range(2):
        j = C_PER_W - 2 + b
        cid = base_cid + j
        h = lax.shift_right_logical(cid, 7)
        tb = cid & 127
        pltpu.make_async_copy(
            obuf.at[j & 1],
            out_hbm.at[h, :, pl.ds(tb * CHUNK, CHUNK)],
            osems[j & 1],
        ).wait()


N_BLK = 7812
N_MAIN = 244


@functools.partial(
    pl.kernel,
    mesh=_mesh,
    out_type=jax.ShapeDtypeStruct((500000, 128), jnp.float32),
    scratch_types=[
        pltpu.VMEM((RING, HIDDEN, CHUNK), jnp.float32),
        pltpu.VMEM((RING, HIDDEN, CHUNK), jnp.float32),
        pltpu.VMEM((32, CHUNK), jnp.float32),
        pltpu.SemaphoreType.DMA,
        pltpu.SemaphoreType.DMA,
        pltpu.SemaphoreType.DMA,
        pltpu.SemaphoreType.DMA,
        pltpu.SemaphoreType.DMA,
        pltpu.SemaphoreType.DMA,
        pltpu.SemaphoreType.DMA,
        pltpu.SemaphoreType.DMA,
    ],
    compiler_params=pltpu.CompilerParams(
        use_tc_tiling_on_sc=True, needs_layout_passes=False
    ),
)
def _sc_pack(tabt_hbm, tail_hbm, out_hbm, gbuf, obuf, tailv,
             gsem0, gsem1, gsem2, gsem3, osem0, osem1, osem2, osem3):
    gsems = (gsem0, gsem1, gsem2, gsem3)
    osems = (osem0, osem1, osem2, osem3)
    wid = lax.axis_index("s") * NC + lax.axis_index("c")

    lane = lax.iota(jnp.int32, 16)
    hi64 = (lane & 1) << 6
    rows = [lane + 16 * t for t in range(8)]
    rowhalf = [(lane >> 1) + 8 * t for t in range(8)]

    def start(k, slot):
        pltpu.async_copy(
            tabt_hbm.at[:, pl.ds(k * CHUNK, CHUNK)], gbuf.at[slot], gsems[slot]
        )

    def wait_in(k, slot):
        pltpu.make_async_copy(
            tabt_hbm.at[:, pl.ds(k * CHUNK, CHUNK)], gbuf.at[slot], gsems[slot]
        ).wait()

    def transpose_block(b):
        @plsc.parallel_loop(0, HIDDEN, step=1, unroll=8)
        def col_body(c):
            svec = (c + lane) & 63
            colv = svec + hi64
            for t in range(8):
                val = plsc.load_gather(gbuf.at[b], [svec, rows[t]])
                plsc.store_scatter(obuf.at[b], [rowhalf[t], colv], val)

    def out_slice(k):
        return out_hbm.at[pl.ds(k * HIDDEN, HIDDEN)]

    for g0 in range(RING - 1):
        start(32 * g0 + wid, g0)

    def group_body(grp, _):
        for b in range(RING):
            g = RING * grp + b
            k = 32 * g + wid

            @pl.when(g + RING - 1 < N_MAIN)
            def _():
                start(k + 32 * (RING - 1), (b + RING - 1) % RING)

            wait_in(k, b)

            @pl.when(g >= RING)
            def _():
                pltpu.make_async_copy(
                    obuf.at[b], out_slice(k - 32 * RING), osems[b]
                ).wait()

            transpose_block(b)
            pltpu.async_copy(obuf.at[b], out_slice(k), osems[b])
        return 0

    lax.fori_loop(0, N_MAIN // RING, group_body, 0)

    for b in range(RING):
        k_last = 32 * (N_MAIN - RING + b) + wid
        pltpu.make_async_copy(obuf.at[b], out_slice(k_last), osems[b]).wait()

    @pl.when(wid < 4)
    def _():
        k = N_MAIN * 32 + wid
        pltpu.sync_copy(tabt_hbm.at[:, pl.ds(k * CHUNK, CHUNK)], gbuf.at[0])
        transpose_block(0)
        pltpu.sync_copy(obuf.at[0], out_slice(k))

    @pl.when(wid == 31)
    def _():
        pltpu.sync_copy(tail_hbm, tailv)
        pltpu.sync_copy(tailv, out_hbm.at[pl.ds(N_BLK * HIDDEN, 32)])


def kernel(table, item_ids):
    tablet = table.T
    tail = table[N_BLK * CHUNK:].reshape(32, 128)
    table2 = _sc_pack(tablet, tail)
    idx2 = item_ids.T.reshape(N_CHUNKS_TOTAL, CHUNK).astype(jnp.int32)
    out = _sc_gather(table2, idx2)
    return jnp.transpose(out, (2, 0, 1))


# speedup vs baseline: 4.3461x; 1.0011x over previous
import functools

import jax
import jax.numpy as jnp
from jax import lax
from jax.experimental import pallas as pl
from jax.experimental.pallas import tpu as pltpu
from jax.experimental.pallas import tpu_sc as plsc

BATCH = 16384
HIST = 50
HIDDEN = 64
B_TOTAL = BATCH * HIST

NC = 2
NS = 16
NW = NC * NS
CHUNK = 128
N_CHUNKS_TOTAL = B_TOTAL // CHUNK
C_PER_W = N_CHUNKS_TOTAL // NW
N_GROUPS = C_PER_W // 2

_mesh = plsc.VectorSubcoreMesh(core_axis_name="c", subcore_axis_name="s")


RING = 4


@functools.partial(
    pl.kernel,
    mesh=_mesh,
    out_type=jax.ShapeDtypeStruct((HIST, HIDDEN, BATCH), jnp.float32),
    scratch_types=[
        pltpu.VMEM((C_PER_W, CHUNK), jnp.int32),
        pltpu.VMEM((RING, CHUNK), jnp.int32),
        pltpu.VMEM((RING, CHUNK, 128), jnp.float32),
        pltpu.VMEM((RING, HIDDEN, CHUNK), jnp.float32),
        pltpu.SemaphoreType.DMA,
        pltpu.SemaphoreType.DMA,
        pltpu.SemaphoreType.DMA,
        pltpu.SemaphoreType.DMA,
        pltpu.SemaphoreType.DMA,
        pltpu.SemaphoreType.DMA,
        pltpu.SemaphoreType.DMA,
        pltpu.SemaphoreType.DMA,
    ],
    compiler_params=pltpu.CompilerParams(
        use_tc_tiling_on_sc=True, needs_layout_passes=False
    ),
)
def _sc_gather(table2_hbm, idx_hbm, out_hbm, idx_v, pid_v, gbuf, obuf,
               gsem0, gsem1, gsem2, gsem3, osem0, osem1, osem2, osem3):
    gsems = (gsem0, gsem1, gsem2, gsem3)
    osems = (osem0, osem1, osem2, osem3)
    wid = lax.axis_index("s") * NC + lax.axis_index("c")
    base_cid = wid * C_PER_W

    pltpu.sync_copy(idx_hbm.at[pl.ds(base_cid, C_PER_W)], idx_v)

    rows = [lax.iota(jnp.int32, 16) + 16 * t for t in range(8)]
    lane = rows[0]

    def start_gather(j, slot):
        for t in range(8):
            v = idx_v[j, pl.ds(16 * t, 16)]
            pid_v[slot, pl.ds(16 * t, 16)] = lax.shift_right_logical(v, 1)
        pltpu.async_copy(
            table2_hbm.at[pid_v.at[slot]], gbuf.at[slot], gsems[slot]
        )

    def wait_gather(slot):
        pltpu.make_async_copy(
            table2_hbm.at[pid_v.at[slot]], gbuf.at[slot], gsems[slot]
        ).wait()

    for j0 in range(RING - 1):
        start_gather(j0, j0)

    def group_body(g, _):
        for b in range(RING):
            j = RING * g + b
            ob = b
            nj = j + RING - 1

            @pl.when(nj < C_PER_W)
            def _():
                start_gather(nj, (b + RING - 1) % RING)

            wait_gather(b)

            cid = base_cid + j
            h = lax.shift_right_logical(cid, 7)
            tb = cid & 127

            @pl.when(j >= RING)
            def _():
                pltpu.make_async_copy(
                    obuf.at[ob],
                    out_hbm.at[h, :, pl.ds(tb * CHUNK, CHUNK)],
                    osems[ob],
                ).wait()

            halves = [(idx_v[j, pl.ds(16 * t, 16)] & 1) << 6 for t in range(8)]

            @plsc.parallel_loop(0, HIDDEN, step=1, unroll=8)
            def col_body(c):
                svec = (c + lane) & 63
                for t in range(8):
                    val = plsc.load_gather(
                        gbuf.at[b], [rows[t], halves[t] + svec]
                    )
                    plsc.store_scatter(obuf.at[ob], [svec, rows[t]], val)

            pltpu.async_copy(
                obuf.at[ob],
                out_hbm.at[h, :, pl.ds(tb * CHUNK, CHUNK)],
                osems[ob],
            )
        return 0

    lax.fori_loop(0, C_PER_W // RING, group_body, 0)

    for b in range(RING):
        j = C_PER_W - RING + b
        cid = base_cid + j
        h = lax.shift_right_logical(cid, 7)
        tb = cid & 127
        pltpu.make_async_copy(
            obuf.at[b],
            out_hbm.at[h, :, pl.ds(tb * CHUNK, CHUNK)],
            osems[b],
        ).wait()


N_BLK = 7812
N_MAIN = 244


@functools.partial(
    pl.kernel,
    mesh=_mesh,
    out_type=jax.ShapeDtypeStruct((500000, 128), jnp.float32),
    scratch_types=[
        pltpu.VMEM((RING, HIDDEN, CHUNK), jnp.float32),
        pltpu.VMEM((RING, HIDDEN, CHUNK), jnp.float32),
        pltpu.VMEM((32, CHUNK), jnp.float32),
        pltpu.SemaphoreType.DMA,
        pltpu.SemaphoreType.DMA,
        pltpu.SemaphoreType.DMA,
        pltpu.SemaphoreType.DMA,
        pltpu.SemaphoreType.DMA,
        pltpu.SemaphoreType.DMA,
        pltpu.SemaphoreType.DMA,
        pltpu.SemaphoreType.DMA,
    ],
    compiler_params=pltpu.CompilerParams(
        use_tc_tiling_on_sc=True, needs_layout_passes=False
    ),
)
def _sc_pack(tabt_hbm, tail_hbm, out_hbm, gbuf, obuf, tailv,
             gsem0, gsem1, gsem2, gsem3, osem0, osem1, osem2, osem3):
    gsems = (gsem0, gsem1, gsem2, gsem3)
    osems = (osem0, osem1, osem2, osem3)
    wid = lax.axis_index("s") * NC + lax.axis_index("c")

    lane = lax.iota(jnp.int32, 16)
    hi64 = (lane & 1) << 6
    rows = [lane + 16 * t for t in range(8)]
    rowhalf = [(lane >> 1) + 8 * t for t in range(8)]

    def start(k, slot):
        pltpu.async_copy(
            tabt_hbm.at[:, pl.ds(k * CHUNK, CHUNK)], gbuf.at[slot], gsems[slot]
        )

    def wait_in(k, slot):
        pltpu.make_async_copy(
            tabt_hbm.at[:, pl.ds(k * CHUNK, CHUNK)], gbuf.at[slot], gsems[slot]
        ).wait()

    def transpose_block(b):
        @plsc.parallel_loop(0, HIDDEN, step=1, unroll=8)
        def col_body(c):
            svec = (c + lane) & 63
            colv = svec + hi64
            for t in range(8):
                val = plsc.load_gather(gbuf.at[b], [svec, rows[t]])
                plsc.store_scatter(obuf.at[b], [rowhalf[t], colv], val)

    def out_slice(k):
        return out_hbm.at[pl.ds(k * HIDDEN, HIDDEN)]

    for g0 in range(RING - 1):
        start(32 * g0 + wid, g0)

    def group_body(grp, _):
        for b in range(RING):
            g = RING * grp + b
            k = 32 * g + wid

            @pl.when(g + RING - 1 < N_MAIN)
            def _():
                start(k + 32 * (RING - 1), (b + RING - 1) % RING)

            wait_in(k, b)

            @pl.when(g >= RING)
            def _():
                pltpu.make_async_copy(
                    obuf.at[b], out_slice(k - 32 * RING), osems[b]
                ).wait()

            transpose_block(b)
            pltpu.async_copy(obuf.at[b], out_slice(k), osems[b])
        return 0

    lax.fori_loop(0, N_MAIN // RING, group_body, 0)

    for b in range(RING):
        k_last = 32 * (N_MAIN - RING + b) + wid
        pltpu.make_async_copy(obuf.at[b], out_slice(k_last), osems[b]).wait()

    @pl.when(wid < 4)
    def _():
        k = N_MAIN * 32 + wid
        pltpu.sync_copy(tabt_hbm.at[:, pl.ds(k * CHUNK, CHUNK)], gbuf.at[0])
        transpose_block(0)
        pltpu.sync_copy(obuf.at[0], out_slice(k))

    @pl.when(wid == 31)
    def _():
        pltpu.sync_copy(tail_hbm, tailv)
        pltpu.sync_copy(tailv, out_hbm.at[pl.ds(N_BLK * HIDDEN, 32)])


def kernel(table, item_ids):
    tablet = table.T
    tail = table[N_BLK * CHUNK:].reshape(32, 128)
    table2 = _sc_pack(tablet, tail)
    idx2 = item_ids.T.reshape(N_CHUNKS_TOTAL, CHUNK).astype(jnp.int32)
    out = _sc_gather(table2, idx2)
    return jnp.transpose(out, (2, 0, 1))
